# baseline (speedup 1.0000x reference)
import functools

import jax
import jax.numpy as jnp
from jax import lax
from jax.experimental import pallas as pl
from jax.experimental.pallas import tpu as pltpu
from jax.experimental.pallas import tpu_sc as plsc

_D_MODEL = 128
_SEQ = 512
_SCALE = 1
_EPS = 1e-6

_NC = 2
_NS = 16
_NW = _NC * _NS

_CH = 128
_HCH = _CH // 2
_NBUF = 4


def _sc_compiler_params():
    cp = pltpu.CompilerParams(use_tc_tiling_on_sc=False)
    if "needs_layout_passes" in pltpu.CompilerParams.__dataclass_fields__:
        import dataclasses

        cp = dataclasses.replace(cp, needs_layout_passes=False)
    return cp


def _sc_gather_pack(table, flat_idx):
    n = flat_idx.shape[0]
    b_per_w = n // _NW
    n_chunks = b_per_w // _CH
    mesh = plsc.VectorSubcoreMesh(core_axis_name="c", subcore_axis_name="s")

    @functools.partial(
        pl.kernel,
        out_type=jax.ShapeDtypeStruct((n // 2, _D_MODEL), jnp.float32),
        mesh=mesh,
        scratch_types=[
            pltpu.VMEM((b_per_w,), jnp.int32),
            [pltpu.VMEM((_CH, _D_MODEL), jnp.float32) for _ in range(_NBUF)],
            [pltpu.VMEM((_HCH, _D_MODEL), jnp.float32) for _ in range(_NBUF)],
            [pltpu.SemaphoreType.DMA for _ in range(_NBUF)],
            [pltpu.SemaphoreType.DMA for _ in range(_NBUF)],
        ],
        compiler_params=_sc_compiler_params(),
    )
    def gather_kernel(table_hbm, idx_hbm, out_hbm, idx_v, bufs, pbufs, gsems, wsems):
        wid = lax.axis_index("s") * _NC + lax.axis_index("c")
        base = wid * b_per_w
        pbase = base // 2
        pltpu.sync_copy(idx_hbm.at[pl.ds(base, b_per_w)], idx_v)

        def start_gather(c, b):
            pltpu.async_copy(
                table_hbm.at[idx_v.at[pl.ds(c * _CH, _CH)]], bufs[b], gsems[b]
            )

        def wait_gather(b):
            pltpu.make_async_copy(
                table_hbm.at[idx_v.at[pl.ds(0, _CH)]], bufs[b], gsems[b]
            ).wait()

        def start_write(c, b):
            pltpu.async_copy(
                pbufs[b], out_hbm.at[pl.ds(pbase + c * _HCH, _HCH)], wsems[b]
            )

        def wait_write(b):
            pltpu.make_async_copy(
                pbufs[b], out_hbm.at[pl.ds(pbase, _HCH)], wsems[b]
            ).wait()

        def convert(b):
            buf, pbuf = bufs[b], pbufs[b]

            @pl.loop(0, _HCH)
            def _(q):
                for k in range(_D_MODEL // 16):
                    a = buf[q, pl.ds(16 * k, 16)]
                    bb = buf[q + _HCH, pl.ds(16 * k, 16)]
                    p = plsc.pack(a, bb, format=plsc.PackFormat.INTERLEAVED)
                    pbuf[q, pl.ds(16 * k, 16)] = plsc.bitcast(p, jnp.float32)

        for b in range(_NBUF):
            start_gather(b, b)

        for b in range(_NBUF):
            wait_gather(b)
            convert(b)
            start_write(b, b)
            start_gather(_NBUF + b, b)

        @pl.loop(_NBUF, n_chunks, step=_NBUF)
        def _(c):
            for b in range(_NBUF):
                wait_gather(b)
                wait_write(b)
                convert(b)
                start_write(c + b, b)

                @pl.when(c + b + _NBUF < n_chunks)
                def _():
                    start_gather(c + b + _NBUF, b)

        for b in range(_NBUF):
            wait_write(b)

    return gather_kernel(table, flat_idx)


_PR = 2048


def _ln_block(w, pos_lo, pos_hi, g, b):
    wi = lax.bitcast_convert_type(w, jnp.uint32)
    lo = lax.bitcast_convert_type(wi << 16, jnp.float32)
    hi = lax.bitcast_convert_type(wi & jnp.uint32(0xFFFF0000), jnp.float32)
    ng = _PR // 256

    def ln(x, pos):
        x = x.reshape(ng, 256, _D_MODEL) * _SCALE + pos[None]
        mean = jnp.mean(x, axis=-1, keepdims=True)
        xc = x - mean
        var = jnp.mean(xc * xc, axis=-1, keepdims=True)
        y = xc * lax.rsqrt(var + _EPS)
        return y * g + b

    y_lo = ln(lo, pos_lo)
    y_hi = ln(hi, pos_hi)
    y_lo4 = y_lo.reshape(ng * 4, 64, _D_MODEL)
    y_hi4 = y_hi.reshape(ng * 4, 64, _D_MODEL)
    return jnp.concatenate([y_lo4, y_hi4], axis=1).reshape(2 * _PR, _D_MODEL)


def _ln_body_first(w_ref, plo_ref, phi_ref, g_ref, b_ref, o_ref):
    o_ref[...] = _ln_block(
        w_ref[...], plo_ref[...], phi_ref[...], g_ref[...], b_ref[...]
    )


def _ln_body_chained(acc_ref, w_ref, plo_ref, phi_ref, g_ref, b_ref, o_ref):
    del acc_ref
    _ln_body_first(w_ref, plo_ref, phi_ref, g_ref, b_ref, o_ref)


def _tc_layernorm_slice(acc, packed, pos_lo, pos_hi, gamma, beta, blk_off, n_rows):
    sl_rows = packed.shape[0]
    grid = (sl_rows // _PR,)
    common_specs = [
        pl.BlockSpec((_PR, _D_MODEL), lambda i: (i, 0)),
        pl.BlockSpec((256, _D_MODEL), lambda i: (0, 0)),
        pl.BlockSpec((256, _D_MODEL), lambda i: (0, 0)),
        pl.BlockSpec((1, _D_MODEL), lambda i: (0, 0)),
        pl.BlockSpec((1, _D_MODEL), lambda i: (0, 0)),
    ]
    out_spec = pl.BlockSpec((2 * _PR, _D_MODEL), lambda i: (blk_off + i, 0))
    out_shape = jax.ShapeDtypeStruct((n_rows, _D_MODEL), jnp.float32)
    if acc is None:
        return pl.pallas_call(
            _ln_body_first,
            grid=grid,
            in_specs=common_specs,
            out_specs=out_spec,
            out_shape=out_shape,
        )(packed, pos_lo, pos_hi, gamma, beta)
    return pl.pallas_call(
        _ln_body_chained,
        grid=grid,
        in_specs=[pl.BlockSpec(memory_space=pl.ANY)] + common_specs,
        out_specs=out_spec,
        out_shape=out_shape,
        input_output_aliases={0: 0},
    )(acc, packed, pos_lo, pos_hi, gamma, beta)


_N_SLICES = 8


@jax.jit
def kernel(inputs, token_table, pos_table, gamma, beta):
    batch, seq = inputs.shape
    n = batch * seq
    flat_idx = inputs.reshape(-1)
    gamma2 = gamma.reshape(1, _D_MODEL)
    beta2 = beta.reshape(1, _D_MODEL)

    m = jnp.arange(256)
    lo_idx = (2 * m - (m % 64)) % _SEQ
    pos_lo = jnp.take(pos_table, lo_idx, axis=0)
    pos_hi = jnp.take(pos_table, (lo_idx + 64) % _SEQ, axis=0)

    sl_n = n // _N_SLICES
    packed = [
        _sc_gather_pack(
            token_table, lax.dynamic_slice(flat_idx, (s * sl_n,), (sl_n,))
        )
        for s in range(_N_SLICES)
    ]
    blocks_per_slice = (sl_n // 2) // _PR
    acc = None
    for s in range(_N_SLICES):
        acc = _tc_layernorm_slice(
            acc, packed[s], pos_lo, pos_hi, gamma2, beta2,
            s * blocks_per_slice, n,
        )
    return acc.reshape(batch, seq, _D_MODEL)


# speedup vs baseline: 1.0491x; 1.0491x over previous
import functools

import jax
import jax.numpy as jnp
from jax import lax
from jax.experimental import pallas as pl
from jax.experimental.pallas import tpu as pltpu
from jax.experimental.pallas import tpu_sc as plsc

_D_MODEL = 128
_SEQ = 512
_SCALE = 1
_EPS = 1e-6

_NC = 2
_NS = 16
_NW = _NC * _NS

_CH = 128
_HCH = _CH // 2
_NBUF = 4


def _sc_compiler_params():
    cp = pltpu.CompilerParams(use_tc_tiling_on_sc=False)
    if "needs_layout_passes" in pltpu.CompilerParams.__dataclass_fields__:
        import dataclasses

        cp = dataclasses.replace(cp, needs_layout_passes=False)
    return cp


def _sc_gather_pack(table, flat_idx):
    n = flat_idx.shape[0]
    b_per_w = n // _NW
    n_chunks = b_per_w // _CH
    mesh = plsc.VectorSubcoreMesh(core_axis_name="c", subcore_axis_name="s")

    @functools.partial(
        pl.kernel,
        out_type=jax.ShapeDtypeStruct((n // 2, _D_MODEL), jnp.float32),
        mesh=mesh,
        scratch_types=[
            pltpu.VMEM((b_per_w,), jnp.int32),
            [pltpu.VMEM((_CH, _D_MODEL), jnp.float32) for _ in range(_NBUF)],
            [pltpu.VMEM((_HCH, _D_MODEL), jnp.float32) for _ in range(_NBUF)],
            [pltpu.SemaphoreType.DMA for _ in range(_NBUF)],
            [pltpu.SemaphoreType.DMA for _ in range(_NBUF)],
        ],
        compiler_params=_sc_compiler_params(),
    )
    def gather_kernel(table_hbm, idx_hbm, out_hbm, idx_v, bufs, pbufs, gsems, wsems):
        wid = lax.axis_index("s") * _NC + lax.axis_index("c")
        base = wid * b_per_w
        pbase = base // 2
        pltpu.sync_copy(idx_hbm.at[pl.ds(base, b_per_w)], idx_v)

        def start_gather(c, b):
            pltpu.async_copy(
                table_hbm.at[idx_v.at[pl.ds(c * _CH, _CH)]], bufs[b], gsems[b]
            )

        def wait_gather(b):
            pltpu.make_async_copy(
                table_hbm.at[idx_v.at[pl.ds(0, _CH)]], bufs[b], gsems[b]
            ).wait()

        def start_write(c, b):
            pltpu.async_copy(
                pbufs[b], out_hbm.at[pl.ds(pbase + c * _HCH, _HCH)], wsems[b]
            )

        def wait_write(b):
            pltpu.make_async_copy(
                pbufs[b], out_hbm.at[pl.ds(pbase, _HCH)], wsems[b]
            ).wait()

        def convert(b):
            buf, pbuf = bufs[b], pbufs[b]

            @pl.loop(0, _HCH)
            def _(q):
                for k in range(_D_MODEL // 16):
                    a = buf[q, pl.ds(16 * k, 16)]
                    bb = buf[q + _HCH, pl.ds(16 * k, 16)]
                    p = plsc.pack(a, bb, format=plsc.PackFormat.INTERLEAVED)
                    pbuf[q, pl.ds(16 * k, 16)] = plsc.bitcast(p, jnp.float32)

        for b in range(_NBUF):
            start_gather(b, b)

        for b in range(_NBUF):
            wait_gather(b)
            convert(b)
            start_write(b, b)
            start_gather(_NBUF + b, b)

        @pl.loop(_NBUF, n_chunks, step=_NBUF)
        def _(c):
            for b in range(_NBUF):
                wait_gather(b)
                wait_write(b)
                convert(b)
                start_write(c + b, b)

                @pl.when(c + b + _NBUF < n_chunks)
                def _():
                    start_gather(c + b + _NBUF, b)

        for b in range(_NBUF):
            wait_write(b)

    return gather_kernel(table, flat_idx)


_PR = 4096


def _ln_block(w, pos_lo, pos_hi, g, b):
    wi = lax.bitcast_convert_type(w, jnp.uint32)
    lo = lax.bitcast_convert_type(wi << 16, jnp.float32)
    hi = lax.bitcast_convert_type(wi & jnp.uint32(0xFFFF0000), jnp.float32)
    ng = _PR // 256

    def ln(x, pos):
        x = x.reshape(ng, 256, _D_MODEL) * _SCALE + pos[None]
        mean = jnp.mean(x, axis=-1, keepdims=True)
        xc = x - mean
        var = jnp.mean(xc * xc, axis=-1, keepdims=True)
        y = xc * lax.rsqrt(var + _EPS)
        return y * g + b

    y_lo = ln(lo, pos_lo)
    y_hi = ln(hi, pos_hi)
    y_lo4 = y_lo.reshape(ng * 4, 64, _D_MODEL)
    y_hi4 = y_hi.reshape(ng * 4, 64, _D_MODEL)
    return jnp.concatenate([y_lo4, y_hi4], axis=1).reshape(2 * _PR, _D_MODEL)


def _ln_body_first(w_ref, plo_ref, phi_ref, g_ref, b_ref, o_ref):
    o_ref[...] = _ln_block(
        w_ref[...], plo_ref[...], phi_ref[...], g_ref[...], b_ref[...]
    )


def _ln_body_chained(acc_ref, w_ref, plo_ref, phi_ref, g_ref, b_ref, o_ref):
    del acc_ref
    _ln_body_first(w_ref, plo_ref, phi_ref, g_ref, b_ref, o_ref)


def _tc_layernorm_slice(acc, packed, pos_lo, pos_hi, gamma, beta, blk_off, n_rows):
    sl_rows = packed.shape[0]
    grid = (sl_rows // _PR,)
    common_specs = [
        pl.BlockSpec((_PR, _D_MODEL), lambda i: (i, 0)),
        pl.BlockSpec((256, _D_MODEL), lambda i: (0, 0)),
        pl.BlockSpec((256, _D_MODEL), lambda i: (0, 0)),
        pl.BlockSpec((1, _D_MODEL), lambda i: (0, 0)),
        pl.BlockSpec((1, _D_MODEL), lambda i: (0, 0)),
    ]
    out_spec = pl.BlockSpec((2 * _PR, _D_MODEL), lambda i: (blk_off + i, 0))
    out_shape = jax.ShapeDtypeStruct((n_rows, _D_MODEL), jnp.float32)
    if acc is None:
        return pl.pallas_call(
            _ln_body_first,
            grid=grid,
            in_specs=common_specs,
            out_specs=out_spec,
            out_shape=out_shape,
        )(packed, pos_lo, pos_hi, gamma, beta)
    return pl.pallas_call(
        _ln_body_chained,
        grid=grid,
        in_specs=[pl.BlockSpec(memory_space=pl.ANY)] + common_specs,
        out_specs=out_spec,
        out_shape=out_shape,
        input_output_aliases={0: 0},
    )(acc, packed, pos_lo, pos_hi, gamma, beta)


_N_SLICES = 4


@jax.jit
def kernel(inputs, token_table, pos_table, gamma, beta):
    batch, seq = inputs.shape
    n = batch * seq
    flat_idx = inputs.reshape(-1)
    gamma2 = gamma.reshape(1, _D_MODEL)
    beta2 = beta.reshape(1, _D_MODEL)

    m = jnp.arange(256)
    lo_idx = (2 * m - (m % 64)) % _SEQ
    pos_lo = jnp.take(pos_table, lo_idx, axis=0)
    pos_hi = jnp.take(pos_table, (lo_idx + 64) % _SEQ, axis=0)

    sl_n = n // _N_SLICES
    packed = [
        _sc_gather_pack(
            token_table, lax.dynamic_slice(flat_idx, (s * sl_n,), (sl_n,))
        )
        for s in range(_N_SLICES)
    ]
    blocks_per_slice = (sl_n // 2) // _PR
    acc = None
    for s in range(_N_SLICES):
        acc = _tc_layernorm_slice(
            acc, packed[s], pos_lo, pos_hi, gamma2, beta2,
            s * blocks_per_slice, n,
        )
    return acc.reshape(batch, seq, _D_MODEL)


# speedup vs baseline: 1.0654x; 1.0155x over previous
import functools

import jax
import jax.numpy as jnp
from jax import lax
from jax.experimental import pallas as pl
from jax.experimental.pallas import tpu as pltpu
from jax.experimental.pallas import tpu_sc as plsc

_D_MODEL = 128
_SEQ = 512
_SCALE = 1
_EPS = 1e-6

_NC = 2
_NS = 16
_NW = _NC * _NS

_CH = 128
_HCH = _CH // 2
_NBUF = 4


def _sc_compiler_params():
    cp = pltpu.CompilerParams(use_tc_tiling_on_sc=False)
    if "needs_layout_passes" in pltpu.CompilerParams.__dataclass_fields__:
        import dataclasses

        cp = dataclasses.replace(cp, needs_layout_passes=False)
    return cp


def _sc_gather_pack(table, flat_idx):
    n = flat_idx.shape[0]
    b_per_w = n // _NW
    n_chunks = b_per_w // _CH
    mesh = plsc.VectorSubcoreMesh(core_axis_name="c", subcore_axis_name="s")

    @functools.partial(
        pl.kernel,
        out_type=jax.ShapeDtypeStruct((n // 2, _D_MODEL), jnp.float32),
        mesh=mesh,
        scratch_types=[
            pltpu.VMEM((b_per_w,), jnp.int32),
            [pltpu.VMEM((_CH, _D_MODEL), jnp.float32) for _ in range(_NBUF)],
            [pltpu.VMEM((_HCH, _D_MODEL), jnp.float32) for _ in range(_NBUF)],
            [pltpu.SemaphoreType.DMA for _ in range(_NBUF)],
            [pltpu.SemaphoreType.DMA for _ in range(_NBUF)],
        ],
        compiler_params=_sc_compiler_params(),
    )
    def gather_kernel(table_hbm, idx_hbm, out_hbm, idx_v, bufs, pbufs, gsems, wsems):
        wid = lax.axis_index("s") * _NC + lax.axis_index("c")
        base = wid * b_per_w
        pbase = base // 2
        pltpu.sync_copy(idx_hbm.at[pl.ds(base, b_per_w)], idx_v)

        def start_gather(c, b):
            pltpu.async_copy(
                table_hbm.at[idx_v.at[pl.ds(c * _CH, _CH)]], bufs[b], gsems[b]
            )

        def wait_gather(b):
            pltpu.make_async_copy(
                table_hbm.at[idx_v.at[pl.ds(0, _CH)]], bufs[b], gsems[b]
            ).wait()

        def start_write(c, b):
            pltpu.async_copy(
                pbufs[b], out_hbm.at[pl.ds(pbase + c * _HCH, _HCH)], wsems[b]
            )

        def wait_write(b):
            pltpu.make_async_copy(
                pbufs[b], out_hbm.at[pl.ds(pbase, _HCH)], wsems[b]
            ).wait()

        def convert(b):
            buf, pbuf = bufs[b], pbufs[b]

            @pl.loop(0, _HCH)
            def _(q):
                for k in range(_D_MODEL // 16):
                    a = buf[q, pl.ds(16 * k, 16)]
                    bb = buf[q + _HCH, pl.ds(16 * k, 16)]
                    p = plsc.pack(a, bb, format=plsc.PackFormat.INTERLEAVED)
                    pbuf[q, pl.ds(16 * k, 16)] = plsc.bitcast(p, jnp.float32)

        for b in range(_NBUF):
            start_gather(b, b)

        for b in range(_NBUF):
            wait_gather(b)
            convert(b)
            start_write(b, b)
            start_gather(_NBUF + b, b)

        @pl.loop(_NBUF, n_chunks, step=_NBUF)
        def _(c):
            for b in range(_NBUF):
                wait_gather(b)
                wait_write(b)
                convert(b)
                start_write(c + b, b)

                @pl.when(c + b + _NBUF < n_chunks)
                def _():
                    start_gather(c + b + _NBUF, b)

        for b in range(_NBUF):
            wait_write(b)

    return gather_kernel(table, flat_idx)


_PR = 8192


def _ln_block(w, pos_lo, pos_hi, g, b):
    wi = lax.bitcast_convert_type(w, jnp.uint32)
    lo = lax.bitcast_convert_type(wi << 16, jnp.float32)
    hi = lax.bitcast_convert_type(wi & jnp.uint32(0xFFFF0000), jnp.float32)
    ng = _PR // 256

    def ln(x, pos):
        x = x.reshape(ng, 256, _D_MODEL) * _SCALE + pos[None]
        mean = jnp.mean(x, axis=-1, keepdims=True)
        xc = x - mean
        var = jnp.mean(xc * xc, axis=-1, keepdims=True)
        y = xc * lax.rsqrt(var + _EPS)
        return y * g + b

    y_lo = ln(lo, pos_lo)
    y_hi = ln(hi, pos_hi)
    y_lo4 = y_lo.reshape(ng * 4, 64, _D_MODEL)
    y_hi4 = y_hi.reshape(ng * 4, 64, _D_MODEL)
    return jnp.concatenate([y_lo4, y_hi4], axis=1).reshape(2 * _PR, _D_MODEL)


def _ln_body_first(w_ref, plo_ref, phi_ref, g_ref, b_ref, o_ref):
    o_ref[...] = _ln_block(
        w_ref[...], plo_ref[...], phi_ref[...], g_ref[...], b_ref[...]
    )


def _ln_body_chained(acc_ref, w_ref, plo_ref, phi_ref, g_ref, b_ref, o_ref):
    del acc_ref
    _ln_body_first(w_ref, plo_ref, phi_ref, g_ref, b_ref, o_ref)


def _tc_layernorm_slice(acc, packed, pos_lo, pos_hi, gamma, beta, blk_off, n_rows):
    sl_rows = packed.shape[0]
    grid = (sl_rows // _PR,)
    common_specs = [
        pl.BlockSpec((_PR, _D_MODEL), lambda i: (i, 0)),
        pl.BlockSpec((256, _D_MODEL), lambda i: (0, 0)),
        pl.BlockSpec((256, _D_MODEL), lambda i: (0, 0)),
        pl.BlockSpec((1, _D_MODEL), lambda i: (0, 0)),
        pl.BlockSpec((1, _D_MODEL), lambda i: (0, 0)),
    ]
    out_spec = pl.BlockSpec((2 * _PR, _D_MODEL), lambda i: (blk_off + i, 0))
    out_shape = jax.ShapeDtypeStruct((n_rows, _D_MODEL), jnp.float32)
    if acc is None:
        return pl.pallas_call(
            _ln_body_first,
            grid=grid,
            in_specs=common_specs,
            out_specs=out_spec,
            out_shape=out_shape,
        )(packed, pos_lo, pos_hi, gamma, beta)
    return pl.pallas_call(
        _ln_body_chained,
        grid=grid,
        in_specs=[pl.BlockSpec(memory_space=pl.ANY)] + common_specs,
        out_specs=out_spec,
        out_shape=out_shape,
        input_output_aliases={0: 0},
    )(acc, packed, pos_lo, pos_hi, gamma, beta)


_N_SLICES = 4


@jax.jit
def kernel(inputs, token_table, pos_table, gamma, beta):
    batch, seq = inputs.shape
    n = batch * seq
    flat_idx = inputs.reshape(-1)
    gamma2 = gamma.reshape(1, _D_MODEL)
    beta2 = beta.reshape(1, _D_MODEL)

    m = jnp.arange(256)
    lo_idx = (2 * m - (m % 64)) % _SEQ
    pos_lo = jnp.take(pos_table, lo_idx, axis=0)
    pos_hi = jnp.take(pos_table, (lo_idx + 64) % _SEQ, axis=0)

    sl_n = n // _N_SLICES
    packed = [
        _sc_gather_pack(
            token_table, lax.dynamic_slice(flat_idx, (s * sl_n,), (sl_n,))
        )
        for s in range(_N_SLICES)
    ]
    blocks_per_slice = (sl_n // 2) // _PR
    acc = None
    for s in range(_N_SLICES):
        acc = _tc_layernorm_slice(
            acc, packed[s], pos_lo, pos_hi, gamma2, beta2,
            s * blocks_per_slice, n,
        )
    return acc.reshape(batch, seq, _D_MODEL)
